# baseline (speedup 1.0000x reference)
import functools

import jax
import jax.numpy as jnp
from jax import lax
from jax.experimental import pallas as pl
from jax.experimental.pallas import tpu as pltpu
from jax.experimental.pallas import tpu_sc as plsc

NC = 2
NS = 16
NW = NC * NS
L = 16


def _spline_sc(x, t, aux, c0f, c1f, c2f, hmf, *, n, chunk):
    B = x.shape[0]
    ni = n - 1
    per_w = B // NW
    n_chunks = per_w // chunk
    mesh = plsc.VectorSubcoreMesh(
        core_axis_name="c", subcore_axis_name="s",
        num_cores=NC, num_subcores=NS)

    @functools.partial(
        pl.kernel,
        out_type=jax.ShapeDtypeStruct((B // 128, 4, 128), jnp.float32),
        mesh=mesh,
        compiler_params=pltpu.CompilerParams(
            needs_layout_passes=False, use_tc_tiling_on_sc=False),
        scratch_types=[
            pltpu.VMEM((n * 3,), jnp.float32),
            pltpu.VMEM((n * 3,), jnp.float32),
            pltpu.VMEM((n * 3,), jnp.float32),
            pltpu.VMEM(((n - 1) * 36,), jnp.float32),
            pltpu.VMEM((19 * n,), jnp.float32),
            pltpu.VMEM((2 * L,), jnp.float32),
            pltpu.VMEM((chunk,), jnp.float32),
            pltpu.VMEM((chunk // 128, 4, 128), jnp.float32),
        ],
    )
    def sck(x_hbm, t_hbm, aux_hbm, c0_hbm, c1_hbm, c2_hbm, hm_hbm, out_hbm,
            c0_v, c1_v, c2_v, hm_v, tbl_v, aux_v, xb, ob):
        iota = lax.iota(jnp.int32, L)

        pltpu.sync_copy(t_hbm, tbl_v.at[pl.ds(0, n)])
        pltpu.sync_copy(aux_hbm, aux_v)
        pltpu.sync_copy(c0_hbm, c0_v)
        pltpu.sync_copy(c1_hbm, c1_v)
        pltpu.sync_copy(c2_hbm, c2_v)
        pltpu.sync_copy(hm_hbm, hm_v)

        cs = (c0_v, c1_v, c2_v)
        for half in range(2):
            kv = iota + half * L
            kc = jnp.minimum(kv, ni - 1)
            h_base = kc * 36
            c_lo = kc * 3
            c_hi = c_lo + 3
            for jdeg in range(6):
                for d in range(3):
                    acc = jnp.zeros((L,), jnp.float32)
                    for m in range(6):
                        hv = plsc.load_gather(hm_v, [h_base + (jdeg * 6 + m)])
                        cb = (c_lo if m < 3 else c_hi) + d
                        cv = plsc.load_gather(cs[m % 3], [cb])
                        acc = acc + hv * cv
                    tbl_v[pl.ds((1 + jdeg * 3 + d) * n + half * L, L)] = acc

        t0v = aux_v[pl.ds(0, L)]
        idtv = aux_v[pl.ds(L, L)]

        wid = lax.axis_index("s") * NC + lax.axis_index("c")
        base = wid * per_w

        def chunk_body(g, carry):
            off = base + g * chunk
            pltpu.sync_copy(x_hbm.at[pl.ds(off, chunk)], xb)

            @plsc.parallel_loop(0, chunk // L, unroll=8)
            def vbody(v):
                xv = xb[pl.ds(v * L, L)]
                u = (xv - t0v) * idtv
                i0 = jnp.clip(u.astype(jnp.int32), 0, ni - 1)
                tlo = plsc.load_gather(tbl_v, [i0])
                thi = plsc.load_gather(tbl_v, [i0 + 1])
                up = jnp.where(xv >= thi, 1, 0).astype(jnp.int32)
                dn = jnp.where(xv < tlo, 1, 0).astype(jnp.int32)
                i = jnp.clip(i0 + up - dn, 0, ni - 1)
                tsel = plsc.load_gather(tbl_v, [i])
                f = xv - tsel
                cb = v // 8
                il = (v % 8) * L
                for d in range(3):
                    acc = plsc.load_gather(tbl_v, [i + (1 + 5 * 3 + d) * n])
                    for jdeg in range(4, -1, -1):
                        cj = plsc.load_gather(tbl_v, [i + (1 + jdeg * 3 + d) * n])
                        acc = acc * f + cj
                    ob[cb, d, pl.ds(il, L)] = acc

            pltpu.sync_copy(
                ob, out_hbm.at[pl.ds(off // 128, chunk // 128)])
            return carry

        lax.fori_loop(0, n_chunks, chunk_body, 0)

    out = sck(x, t, aux, c0f, c1f, c2f, hmf)
    return jnp.transpose(out[:, :3, :], (0, 2, 1)).reshape(B, 3)


def kernel(x, t, c0, c1, c2, hmat):
    n = t.shape[0]
    B = x.shape[0]
    assert B % (NW * L) == 0
    chunk = 16384
    while B // NW % chunk:
        chunk //= 2
    aux = jnp.concatenate([
        jnp.broadcast_to(t[0], (L,)),
        jnp.broadcast_to(1.0 / (t[1] - t[0]), (L,)),
    ]).astype(jnp.float32)
    return _spline_sc(
        x, t, aux,
        c0.reshape(-1), c1.reshape(-1), c2.reshape(-1), hmat.reshape(-1),
        n=n, chunk=chunk)


# speedup vs baseline: 1.6346x; 1.6346x over previous
import functools

import jax
import jax.numpy as jnp
from jax import lax
from jax.experimental import pallas as pl
from jax.experimental.pallas import tpu as pltpu
from jax.experimental.pallas import tpu_sc as plsc

NC = 2
NS = 16
NW = NC * NS
L = 16


def _spline_sc(x, t, aux, c0f, c1f, c2f, hmf, *, n, chunk):
    B = x.shape[0]
    ni = n - 1
    per_w = B // NW
    n_chunks = per_w // chunk
    mesh = plsc.VectorSubcoreMesh(
        core_axis_name="c", subcore_axis_name="s",
        num_cores=NC, num_subcores=NS)

    @functools.partial(
        pl.kernel,
        out_type=jax.ShapeDtypeStruct((B // 128, 4, 128), jnp.float32),
        mesh=mesh,
        compiler_params=pltpu.CompilerParams(
            needs_layout_passes=False, use_tc_tiling_on_sc=False),
        scratch_types=[
            pltpu.VMEM((n * 3,), jnp.float32),
            pltpu.VMEM((n * 3,), jnp.float32),
            pltpu.VMEM((n * 3,), jnp.float32),
            pltpu.VMEM(((n - 1) * 36,), jnp.float32),
            pltpu.VMEM((19 * n,), jnp.float32),
            pltpu.VMEM((2 * L,), jnp.float32),
            pltpu.VMEM((chunk,), jnp.float32),
            pltpu.VMEM((chunk // 128, 4, 128), jnp.float32),
        ],
    )
    def sck(x_hbm, t_hbm, aux_hbm, c0_hbm, c1_hbm, c2_hbm, hm_hbm, out_hbm,
            c0_v, c1_v, c2_v, hm_v, tbl_v, aux_v, xb, ob):
        iota = lax.iota(jnp.int32, L)

        pltpu.sync_copy(t_hbm, tbl_v.at[pl.ds(0, n)])
        pltpu.sync_copy(aux_hbm, aux_v)
        pltpu.sync_copy(c0_hbm, c0_v)
        pltpu.sync_copy(c1_hbm, c1_v)
        pltpu.sync_copy(c2_hbm, c2_v)
        pltpu.sync_copy(hm_hbm, hm_v)

        cs = (c0_v, c1_v, c2_v)
        for half in range(2):
            kv = iota + half * L
            kc = jnp.minimum(kv, ni - 1)
            h_base = kc * 36
            c_lo = kc * 3
            c_hi = c_lo + 3
            for jdeg in range(6):
                for d in range(3):
                    acc = jnp.zeros((L,), jnp.float32)
                    for m in range(6):
                        hv = plsc.load_gather(hm_v, [h_base + (jdeg * 6 + m)])
                        cb = (c_lo if m < 3 else c_hi) + d
                        cv = plsc.load_gather(cs[m % 3], [cb])
                        acc = acc + hv * cv
                    tbl_v[pl.ds((1 + jdeg * 3 + d) * n + half * L, L)] = acc

        t0v = aux_v[pl.ds(0, L)]
        idtv = aux_v[pl.ds(L, L)]

        wid = lax.axis_index("s") * NC + lax.axis_index("c")
        base = wid * per_w

        def chunk_body(g, carry):
            off = base + g * chunk
            pltpu.sync_copy(x_hbm.at[pl.ds(off, chunk)], xb)

            @plsc.parallel_loop(0, chunk // L, unroll=4)
            def vbody(v):
                xv = xb[pl.ds(v * L, L)]
                u = (xv - t0v) * idtv
                i0 = jnp.clip(u.astype(jnp.int32), 0, ni - 1)
                tlo = plsc.load_gather(tbl_v, [i0])
                thi = plsc.load_gather(tbl_v, [i0 + 1])
                up = jnp.where(xv >= thi, 1, 0).astype(jnp.int32)
                dn = jnp.where(xv < tlo, 1, 0).astype(jnp.int32)
                i = jnp.clip(i0 + up - dn, 0, ni - 1)
                tsel = plsc.load_gather(tbl_v, [i])
                f = xv - tsel
                cb = v // 8
                il = (v % 8) * L
                for d in range(3):
                    acc = plsc.load_gather(tbl_v, [i + (1 + 5 * 3 + d) * n])
                    for jdeg in range(4, -1, -1):
                        cj = plsc.load_gather(tbl_v, [i + (1 + jdeg * 3 + d) * n])
                        acc = acc * f + cj
                    ob[cb, d, pl.ds(il, L)] = acc

            pltpu.sync_copy(
                ob, out_hbm.at[pl.ds(off // 128, chunk // 128)])
            return carry

        lax.fori_loop(0, n_chunks, chunk_body, 0)

    out = sck(x, t, aux, c0f, c1f, c2f, hmf)
    return jnp.transpose(out[:, :3, :], (0, 2, 1)).reshape(B, 3)


def kernel(x, t, c0, c1, c2, hmat):
    n = t.shape[0]
    B = x.shape[0]
    assert B % (NW * L) == 0
    chunk = 16384
    while B // NW % chunk:
        chunk //= 2
    aux = jnp.concatenate([
        jnp.broadcast_to(t[0], (L,)),
        jnp.broadcast_to(1.0 / (t[1] - t[0]), (L,)),
    ]).astype(jnp.float32)
    return _spline_sc(
        x, t, aux,
        c0.reshape(-1), c1.reshape(-1), c2.reshape(-1), hmat.reshape(-1),
        n=n, chunk=chunk)


# speedup vs baseline: 1.8794x; 1.1498x over previous
import functools

import jax
import jax.numpy as jnp
from jax import lax
from jax.experimental import pallas as pl
from jax.experimental.pallas import tpu as pltpu
from jax.experimental.pallas import tpu_sc as plsc

NC = 2
NS = 16
NW = NC * NS
L = 16


def _spline_sc(x, aux, c0f, c1f, c2f, hmf, *, n, chunk):
    B = x.shape[0]
    ni = n - 1
    per_w = B // NW
    n_chunks = per_w // chunk
    nblk = chunk // 128
    mesh = plsc.VectorSubcoreMesh(
        core_axis_name="c", subcore_axis_name="s",
        num_cores=NC, num_subcores=NS)

    @functools.partial(
        pl.kernel,
        out_type=jax.ShapeDtypeStruct((B // 128, 4, 128), jnp.float32),
        mesh=mesh,
        compiler_params=pltpu.CompilerParams(
            needs_layout_passes=False, use_tc_tiling_on_sc=False),
        scratch_types=[
            pltpu.VMEM((n * 3,), jnp.float32),
            pltpu.VMEM((n * 3,), jnp.float32),
            pltpu.VMEM((n * 3,), jnp.float32),
            pltpu.VMEM(((n - 1) * 36,), jnp.float32),
            pltpu.VMEM((18 * n,), jnp.float32),
            pltpu.VMEM((3 * L,), jnp.float32),
            pltpu.VMEM((chunk,), jnp.float32),
            pltpu.VMEM((chunk,), jnp.float32),
            pltpu.VMEM((nblk, 4, 128), jnp.float32),
            pltpu.VMEM((nblk, 4, 128), jnp.float32),
            pltpu.SemaphoreType.DMA,
            pltpu.SemaphoreType.DMA,
            pltpu.SemaphoreType.DMA,
            pltpu.SemaphoreType.DMA,
        ],
    )
    def sck(x_hbm, aux_hbm, c0_hbm, c1_hbm, c2_hbm, hm_hbm, out_hbm,
            c0_v, c1_v, c2_v, hm_v, tbl_v, aux_v, xb0, xb1, ob0, ob1,
            isem0, isem1, osem0, osem1):
        iota = lax.iota(jnp.int32, L)
        xbs, obs = (xb0, xb1), (ob0, ob1)
        isems, osems = (isem0, isem1), (osem0, osem1)

        pltpu.sync_copy(aux_hbm, aux_v)
        pltpu.sync_copy(c0_hbm, c0_v)
        pltpu.sync_copy(c1_hbm, c1_v)
        pltpu.sync_copy(c2_hbm, c2_v)
        pltpu.sync_copy(hm_hbm, hm_v)

        cs = (c0_v, c1_v, c2_v)
        for half in range(2):
            kv = iota + half * L
            kc = jnp.minimum(kv, ni - 1)
            h_base = kc * 36
            c_lo = kc * 3
            c_hi = c_lo + 3
            for jdeg in range(6):
                for d in range(3):
                    acc = jnp.zeros((L,), jnp.float32)
                    for m in range(6):
                        hv = plsc.load_gather(hm_v, [h_base + (jdeg * 6 + m)])
                        cb = (c_lo if m < 3 else c_hi) + d
                        cv = plsc.load_gather(cs[m % 3], [cb])
                        acc = acc + hv * cv
                    tbl_v[pl.ds((jdeg * 3 + d) * n + half * L, L)] = acc

        t0v = aux_v[pl.ds(0, L)]
        idtv = aux_v[pl.ds(L, L)]
        dtv = aux_v[pl.ds(2 * L, L)]

        wid = lax.axis_index("s") * NC + lax.axis_index("c")
        base = wid * per_w

        def compute(xb, ob):
            @plsc.parallel_loop(0, chunk // L, unroll=4)
            def vbody(v):
                xv = xb[pl.ds(v * L, L)]
                s = xv - t0v
                u = s * idtv
                i0 = jnp.clip(u.astype(jnp.int32), 0, ni - 1)
                i0f = i0.astype(jnp.float32)
                tlo = t0v + i0f * dtv
                thi = tlo + dtv
                up = jnp.where(xv >= thi, 1, 0).astype(jnp.int32)
                dn = jnp.where(xv < tlo, 1, 0).astype(jnp.int32)
                i = jnp.clip(i0 + up - dn, 0, ni - 1)
                f = s - i.astype(jnp.float32) * dtv
                cb = v // 8
                il = (v % 8) * L
                for d in range(3):
                    acc = plsc.load_gather(tbl_v, [i + (5 * 3 + d) * n])
                    for jdeg in range(4, -1, -1):
                        cj = plsc.load_gather(tbl_v, [i + (jdeg * 3 + d) * n])
                        acc = acc * f + cj
                    ob[cb, d, pl.ds(il, L)] = acc

        def in_copy(g):
            return pltpu.make_async_copy(
                x_hbm.at[pl.ds(base + g * chunk, chunk)],
                xbs[g % 2], isems[g % 2])

        def out_copy(g):
            return pltpu.make_async_copy(
                obs[g % 2],
                out_hbm.at[pl.ds((base + g * chunk) // 128, nblk)],
                osems[g % 2])

        in_copy(0).start()
        for g in range(n_chunks):
            if g + 1 < n_chunks:
                in_copy(g + 1).start()
            if g >= 2:
                out_copy(g - 2).wait()
            in_copy(g).wait()
            compute(xbs[g % 2], obs[g % 2])
            out_copy(g).start()
        for g in range(max(n_chunks - 2, 0), n_chunks):
            out_copy(g).wait()

    return sck(x, aux, c0f, c1f, c2f, hmf)


def kernel(x, t, c0, c1, c2, hmat):
    n = t.shape[0]
    B = x.shape[0]
    assert B % (NW * 128) == 0
    chunk = 8192
    while B // NW % chunk:
        chunk //= 2
    dt = t[1] - t[0]
    aux = jnp.concatenate([
        jnp.broadcast_to(t[0], (L,)),
        jnp.broadcast_to(1.0 / dt, (L,)),
        jnp.broadcast_to(dt, (L,)),
    ]).astype(jnp.float32)
    out = _spline_sc(
        x, aux,
        c0.reshape(-1), c1.reshape(-1), c2.reshape(-1), hmat.reshape(-1),
        n=n, chunk=chunk)
    return jnp.transpose(out[:, :3, :], (0, 2, 1)).reshape(B, 3)
